# baseline (speedup 1.0000x reference)
import functools

import jax
import jax.numpy as jnp
from jax import lax
from jax.experimental import pallas as pl
from jax.experimental.pallas import tpu as pltpu
from jax.experimental.pallas import tpu_sc as plsc

_B = 3.0
_K = 5
_DIM = 128
_N = 131072

_NC = 2
_NS = 16
_NW = _NC * _NS
_CH = 128
_ROWS_PER_W = _N // _NW

_TAB_ROWS = 32


def _knot_tables(wt, ht, dt):
    ew = jnp.exp(wt - jnp.max(wt, axis=0, keepdims=True))
    widths = ew / jnp.sum(ew, axis=0, keepdims=True) * (2.0 * _B)
    eh = jnp.exp(ht - jnp.max(ht, axis=0, keepdims=True))
    heights = eh / jnp.sum(eh, axis=0, keepdims=True) * (2.0 * _B)
    sp = jnp.maximum(dt, 0.0) + jnp.log1p(jnp.exp(-jnp.abs(dt)))

    negb = jnp.full((_DIM,), -_B, wt.dtype)
    cw = [negb]
    ch = [negb]
    for k in range(_K - 1):
        cw.append(cw[-1] + widths[k])
        ch.append(ch[-1] + heights[k])
    one = jnp.ones((_DIM,), wt.dtype)
    dv = [one] + [sp[k] for k in range(_K - 1)] + [one]
    return widths, heights, cw, ch, dv


def _prep_body(wt_ref, ht_ref, dt_ref, tab_ref):
    wt = wt_ref[...]
    ht = ht_ref[...]
    dt = dt_ref[...]
    widths, heights, cw, ch, dv = _knot_tables(wt, ht, dt)
    rows = []
    rows += [cw[k] for k in range(_K)]
    rows += [1.0 / widths[k] for k in range(_K)]
    rows += [ch[k] for k in range(_K)]
    rows += [heights[k] for k in range(_K)]
    rows += dv
    rows += [cw[k] for k in range(1, _K)]
    rows += [jnp.zeros((_DIM,), wt.dtype)] * 2
    tab_ref[...] = jnp.concatenate([r.reshape(1, _DIM) for r in rows], axis=0)


def _prep_call(wt, ht, dt):
    return pl.pallas_call(
        _prep_body,
        out_shape=jax.ShapeDtypeStruct((_TAB_ROWS, _DIM), wt.dtype),
    )(wt, ht, dt)


_LN2 = 0.6931471805599453
_LOG_C = (
    3.342326879394774e-08, 1.0000030986470898, -0.5000129330593671,
    0.33304812395026007, -0.24911210645450632, 0.20611785239613029,
    -0.18627697325403644, 0.11448435452423278,
)


def _sc_log(x):
    bits = lax.bitcast_convert_type(x, jnp.int32)
    e = lax.shift_right_arithmetic(bits, 23) - 127
    mbits = lax.bitwise_or(
        lax.bitwise_and(bits, 0x007FFFFF), 0x3F800000
    )
    m = lax.bitcast_convert_type(mbits, jnp.float32)
    big = m >= 1.4142135623730951
    m = jnp.where(big, m * 0.5, m)
    radj = jnp.where(big, _LN2, 0.0)
    t = m - 1.0
    p = jnp.float32(_LOG_C[7])
    for c in _LOG_C[6::-1]:
        p = p * t + c
    return e.astype(jnp.float32) * _LN2 + (radj + p)


def _sc_body(u_hbm, tab_hbm, x_hbm, ld_hbm, tab_v, ubuf, xbuf, ldbuf, accbuf,
             *, rows_per_w):
    cid = lax.axis_index("c")
    sid = lax.axis_index("s")
    wid = sid * _NC + cid
    pltpu.sync_copy(tab_hbm, tab_v)
    iota16 = lax.iota(jnp.int32, 16)

    def row_body(r, _):
        acc = jnp.zeros((16,), jnp.float32)
        for j in range(_DIM // 16):
            off = 16 * j
            u_v = ubuf[r, pl.ds(off, 16)]
            uc = jnp.clip(u_v, -_B, _B)
            inside = jnp.abs(u_v) <= _B
            binv = jnp.zeros((16,), jnp.int32)
            for k in range(_K - 1):
                ck = tab_v[26 + k, pl.ds(off, 16)]
                binv = binv + (uc >= ck).astype(jnp.int32)
            dimv = iota16 + off
            xk = plsc.load_gather(tab_v, [binv, dimv])
            rw = plsc.load_gather(tab_v, [binv + 5, dimv])
            yk = plsc.load_gather(tab_v, [binv + 10, dimv])
            hk = plsc.load_gather(tab_v, [binv + 15, dimv])
            dk = plsc.load_gather(tab_v, [binv + 20, dimv])
            dk1 = plsc.load_gather(tab_v, [binv + 21, dimv])
            theta = (uc - xk) * rw
            s = hk * rw
            omt = 1.0 - theta
            t1m = theta * omt
            denom = s + (dk + dk1 - 2.0 * s) * t1m
            rden = 1.0 / denom
            th2 = theta * theta
            x_in = yk + hk * (s * th2 + dk * t1m) * rden
            num = s * s * (dk1 * th2 + 2.0 * s * t1m + dk * omt * omt)
            ld_el = _sc_log(num * rden * rden)
            xbuf[r, pl.ds(off, 16)] = jnp.where(inside, x_in, u_v)
            acc = acc + jnp.where(inside, ld_el, 0.0)
        accbuf[r, :] = acc
        return 0

    def chunk_body(ci, _):
        base = wid * rows_per_w + ci * _CH
        pltpu.sync_copy(u_hbm.at[pl.ds(base, _CH)], ubuf)
        lax.fori_loop(0, _CH, row_body, 0)
        for rg in range(_CH // 16):
            rows = iota16 + (16 * rg)
            total = jnp.zeros((16,), jnp.float32)
            for c in range(16):
                cols = jnp.full((16,), c, jnp.int32)
                total = total + plsc.load_gather(accbuf, [rows, cols])
            ldbuf[pl.ds(16 * rg, 16)] = total
        pltpu.sync_copy(xbuf, x_hbm.at[pl.ds(base, _CH)])
        pltpu.sync_copy(ldbuf, ld_hbm.at[pl.ds(base, _CH)])
        return 0

    lax.fori_loop(0, rows_per_w // _CH, chunk_body, 0)


@functools.cache
def _make_sc_main(n_rows):
    rows_per_w = n_rows // _NW
    body = functools.partial(_sc_body, rows_per_w=rows_per_w)
    return functools.partial(
        pl.kernel,
        mesh=plsc.VectorSubcoreMesh(core_axis_name="c", subcore_axis_name="s"),
        out_type=[
            jax.ShapeDtypeStruct((n_rows, _DIM), jnp.float32),
            jax.ShapeDtypeStruct((n_rows,), jnp.float32),
        ],
        scratch_types=[
            pltpu.VMEM((_TAB_ROWS, _DIM), jnp.float32),
            pltpu.VMEM((_CH, _DIM), jnp.float32),
            pltpu.VMEM((_CH, _DIM), jnp.float32),
            pltpu.VMEM((_CH,), jnp.float32),
            pltpu.VMEM((_CH, 16), jnp.float32),
        ],
        compiler_params=pltpu.CompilerParams(needs_layout_passes=False),
    )(body)


_BR = 4096


def _rqs_block(u, wt, ht, dt):
    widths, heights, cw, ch, dv = _knot_tables(wt, ht, dt)
    cw = cw + [jnp.full((_DIM,), _B, u.dtype)]
    ch = ch + [jnp.full((_DIM,), _B, u.dtype)]
    rw = [1.0 / widths[k] for k in range(_K)]
    hh = [heights[k] for k in range(_K)]

    inside = (u >= -_B) & (u <= _B)
    uc = jnp.clip(u, -_B, _B)

    xk = jnp.broadcast_to(cw[0], u.shape)
    rwk = jnp.broadcast_to(rw[0], u.shape)
    yk = jnp.broadcast_to(ch[0], u.shape)
    hk = jnp.broadcast_to(hh[0], u.shape)
    dk = jnp.broadcast_to(dv[0], u.shape)
    dk1 = jnp.broadcast_to(dv[1], u.shape)
    for k in range(1, _K):
        m = uc >= cw[k]
        xk = jnp.where(m, cw[k], xk)
        rwk = jnp.where(m, rw[k], rwk)
        yk = jnp.where(m, ch[k], yk)
        hk = jnp.where(m, hh[k], hk)
        dk = jnp.where(m, dv[k], dk)
        dk1 = jnp.where(m, dv[k + 1], dk1)

    s = hk * rwk
    theta = (uc - xk) * rwk
    omt = 1.0 - theta
    t1m = theta * omt
    denom = s + (dk1 + dk - 2.0 * s) * t1m
    rden = 1.0 / denom
    x_in = yk + hk * (s * theta * theta + dk * t1m) * rden
    num = s * s * (dk1 * theta * theta + 2.0 * s * t1m + dk * omt * omt)
    logd_in = jnp.log(num * rden * rden)
    x = jnp.where(inside, x_in, u)
    ld = jnp.where(inside, logd_in, 0.0)
    return x, jnp.sum(ld, axis=-1, keepdims=True)


def _tc_body(u_ref, wt_ref, ht_ref, dt_ref, x_ref, ld_ref):
    x, ld = _rqs_block(u_ref[...], wt_ref[...], ht_ref[...], dt_ref[...])
    x_ref[...] = x
    ld_ref[...] = ld


def _tc_call(u, wt, ht, dt, interpret=False, n=None, row_offset=0):
    if n is None:
        n = u.shape[0]
    off_blocks = row_offset // _BR
    x, ld = pl.pallas_call(
        _tc_body,
        grid=(n // _BR,),
        in_specs=[
            pl.BlockSpec((_BR, _DIM), lambda i: (i + off_blocks, 0)),
            pl.BlockSpec((_K, _DIM), lambda i: (0, 0)),
            pl.BlockSpec((_K, _DIM), lambda i: (0, 0)),
            pl.BlockSpec((_K - 1, _DIM), lambda i: (0, 0)),
        ],
        out_specs=[
            pl.BlockSpec((_BR, _DIM), lambda i: (i, 0)),
            pl.BlockSpec((_BR, 1), lambda i: (i, 0)),
        ],
        out_shape=[
            jax.ShapeDtypeStruct((n, _DIM), u.dtype),
            jax.ShapeDtypeStruct((n, 1), u.dtype),
        ],
        interpret=interpret,
    )(u, wt, ht, dt)
    return x, ld.reshape(n)


_N_SC = 8192


@jax.jit
def kernel(u, w, h, d):
    wt, ht, dt = w.T, h.T, d.T
    tab = _prep_call(wt, ht, dt)
    x_sc, ld_sc = _make_sc_main(_N_SC)(u, tab)
    x_tc, ld_tc = _tc_call(u, wt, ht, dt, n=_N - _N_SC, row_offset=_N_SC)
    x = jnp.concatenate([x_sc, x_tc], axis=0)
    ld = jnp.concatenate([ld_sc, ld_tc], axis=0)
    return x, ld


# speedup vs baseline: 1.0236x; 1.0236x over previous
import functools

import jax
import jax.numpy as jnp
from jax import lax
from jax.experimental import pallas as pl
from jax.experimental.pallas import tpu as pltpu
from jax.experimental.pallas import tpu_sc as plsc

_B = 3.0
_K = 5
_DIM = 128
_N = 131072

_NC = 2
_NS = 16
_NW = _NC * _NS
_CH = 128
_ROWS_PER_W = _N // _NW

_TAB_ROWS = 32


def _knot_tables(wt, ht, dt):
    ew = jnp.exp(wt - jnp.max(wt, axis=0, keepdims=True))
    widths = ew / jnp.sum(ew, axis=0, keepdims=True) * (2.0 * _B)
    eh = jnp.exp(ht - jnp.max(ht, axis=0, keepdims=True))
    heights = eh / jnp.sum(eh, axis=0, keepdims=True) * (2.0 * _B)
    sp = jnp.maximum(dt, 0.0) + jnp.log1p(jnp.exp(-jnp.abs(dt)))

    negb = jnp.full((_DIM,), -_B, wt.dtype)
    cw = [negb]
    ch = [negb]
    for k in range(_K - 1):
        cw.append(cw[-1] + widths[k])
        ch.append(ch[-1] + heights[k])
    one = jnp.ones((_DIM,), wt.dtype)
    dv = [one] + [sp[k] for k in range(_K - 1)] + [one]
    return widths, heights, cw, ch, dv


def _prep_body(wt_ref, ht_ref, dt_ref, tab_ref):
    wt = wt_ref[...]
    ht = ht_ref[...]
    dt = dt_ref[...]
    widths, heights, cw, ch, dv = _knot_tables(wt, ht, dt)
    rows = []
    rows += [cw[k] for k in range(_K)]
    rows += [1.0 / widths[k] for k in range(_K)]
    rows += [ch[k] for k in range(_K)]
    rows += [heights[k] for k in range(_K)]
    rows += dv
    rows += [cw[k] for k in range(1, _K)]
    rows += [jnp.zeros((_DIM,), wt.dtype)] * 2
    tab_ref[...] = jnp.concatenate([r.reshape(1, _DIM) for r in rows], axis=0)


def _prep_call(wt, ht, dt):
    return pl.pallas_call(
        _prep_body,
        out_shape=jax.ShapeDtypeStruct((_TAB_ROWS, _DIM), wt.dtype),
    )(wt, ht, dt)


_LN2 = 0.6931471805599453
_LOG_C = (
    3.342326879394774e-08, 1.0000030986470898, -0.5000129330593671,
    0.33304812395026007, -0.24911210645450632, 0.20611785239613029,
    -0.18627697325403644, 0.11448435452423278,
)


def _sc_log(x):
    bits = lax.bitcast_convert_type(x, jnp.int32)
    e = lax.shift_right_arithmetic(bits, 23) - 127
    mbits = lax.bitwise_or(
        lax.bitwise_and(bits, 0x007FFFFF), 0x3F800000
    )
    m = lax.bitcast_convert_type(mbits, jnp.float32)
    big = m >= 1.4142135623730951
    m = jnp.where(big, m * 0.5, m)
    radj = jnp.where(big, _LN2, 0.0)
    t = m - 1.0
    p = jnp.float32(_LOG_C[7])
    for c in _LOG_C[6::-1]:
        p = p * t + c
    return e.astype(jnp.float32) * _LN2 + (radj + p)


def _sc_body(u_hbm, tab_hbm, x_hbm, ld_hbm, tab_v, ubuf, xbuf, ldbuf, accbuf,
             *, rows_per_w):
    cid = lax.axis_index("c")
    sid = lax.axis_index("s")
    wid = sid * _NC + cid
    pltpu.sync_copy(tab_hbm, tab_v)
    iota16 = lax.iota(jnp.int32, 16)

    def row_body(r, _):
        acc = jnp.zeros((16,), jnp.float32)
        for j in range(_DIM // 16):
            off = 16 * j
            u_v = ubuf[r, pl.ds(off, 16)]
            uc = jnp.clip(u_v, -_B, _B)
            inside = jnp.abs(u_v) <= _B
            binv = jnp.zeros((16,), jnp.int32)
            for k in range(_K - 1):
                ck = tab_v[26 + k, pl.ds(off, 16)]
                binv = binv + (uc >= ck).astype(jnp.int32)
            dimv = iota16 + off
            xk = plsc.load_gather(tab_v, [binv, dimv])
            rw = plsc.load_gather(tab_v, [binv + 5, dimv])
            yk = plsc.load_gather(tab_v, [binv + 10, dimv])
            hk = plsc.load_gather(tab_v, [binv + 15, dimv])
            dk = plsc.load_gather(tab_v, [binv + 20, dimv])
            dk1 = plsc.load_gather(tab_v, [binv + 21, dimv])
            theta = (uc - xk) * rw
            s = hk * rw
            omt = 1.0 - theta
            t1m = theta * omt
            denom = s + (dk + dk1 - 2.0 * s) * t1m
            rden = 1.0 / denom
            th2 = theta * theta
            x_in = yk + hk * (s * th2 + dk * t1m) * rden
            num = s * s * (dk1 * th2 + 2.0 * s * t1m + dk * omt * omt)
            ld_el = _sc_log(num * rden * rden)
            xbuf[r, pl.ds(off, 16)] = jnp.where(inside, x_in, u_v)
            acc = acc + jnp.where(inside, ld_el, 0.0)
        accbuf[r, :] = acc
        return 0

    def chunk_body(ci, _):
        base = wid * rows_per_w + ci * _CH
        pltpu.sync_copy(u_hbm.at[pl.ds(base, _CH)], ubuf)
        lax.fori_loop(0, _CH, row_body, 0)
        for rg in range(_CH // 16):
            rows = iota16 + (16 * rg)
            total = jnp.zeros((16,), jnp.float32)
            for c in range(16):
                cols = jnp.full((16,), c, jnp.int32)
                total = total + plsc.load_gather(accbuf, [rows, cols])
            ldbuf[pl.ds(16 * rg, 16)] = total
        pltpu.sync_copy(xbuf, x_hbm.at[pl.ds(base, _CH)])
        pltpu.sync_copy(ldbuf, ld_hbm.at[pl.ds(base, _CH)])
        return 0

    lax.fori_loop(0, rows_per_w // _CH, chunk_body, 0)


@functools.cache
def _make_sc_main(n_rows):
    rows_per_w = n_rows // _NW
    body = functools.partial(_sc_body, rows_per_w=rows_per_w)
    return functools.partial(
        pl.kernel,
        mesh=plsc.VectorSubcoreMesh(core_axis_name="c", subcore_axis_name="s"),
        out_type=[
            jax.ShapeDtypeStruct((n_rows, _DIM), jnp.float32),
            jax.ShapeDtypeStruct((n_rows,), jnp.float32),
        ],
        scratch_types=[
            pltpu.VMEM((_TAB_ROWS, _DIM), jnp.float32),
            pltpu.VMEM((_CH, _DIM), jnp.float32),
            pltpu.VMEM((_CH, _DIM), jnp.float32),
            pltpu.VMEM((_CH,), jnp.float32),
            pltpu.VMEM((_CH, 16), jnp.float32),
        ],
        compiler_params=pltpu.CompilerParams(needs_layout_passes=False),
    )(body)


_BR = 4096


def _rqs_block(u, wt, ht, dt):
    widths, heights, cw, ch, dv = _knot_tables(wt, ht, dt)
    cw = cw + [jnp.full((_DIM,), _B, u.dtype)]
    ch = ch + [jnp.full((_DIM,), _B, u.dtype)]
    rw = [1.0 / widths[k] for k in range(_K)]
    hh = [heights[k] for k in range(_K)]

    inside = (u >= -_B) & (u <= _B)
    uc = jnp.clip(u, -_B, _B)

    xk = jnp.broadcast_to(cw[0], u.shape)
    rwk = jnp.broadcast_to(rw[0], u.shape)
    yk = jnp.broadcast_to(ch[0], u.shape)
    hk = jnp.broadcast_to(hh[0], u.shape)
    dk = jnp.broadcast_to(dv[0], u.shape)
    dk1 = jnp.broadcast_to(dv[1], u.shape)
    for k in range(1, _K):
        m = uc >= cw[k]
        xk = jnp.where(m, cw[k], xk)
        rwk = jnp.where(m, rw[k], rwk)
        yk = jnp.where(m, ch[k], yk)
        hk = jnp.where(m, hh[k], hk)
        dk = jnp.where(m, dv[k], dk)
        dk1 = jnp.where(m, dv[k + 1], dk1)

    s = hk * rwk
    theta = (uc - xk) * rwk
    omt = 1.0 - theta
    t1m = theta * omt
    denom = s + (dk1 + dk - 2.0 * s) * t1m
    rden = 1.0 / denom
    x_in = yk + hk * (s * theta * theta + dk * t1m) * rden
    num = s * s * (dk1 * theta * theta + 2.0 * s * t1m + dk * omt * omt)
    logd_in = jnp.log(num * rden * rden)
    x = jnp.where(inside, x_in, u)
    ld = jnp.where(inside, logd_in, 0.0)
    return x, jnp.sum(ld, axis=-1, keepdims=True)


def _tc_body(u_ref, wt_ref, ht_ref, dt_ref, x_ref, ld_ref):
    x, ld = _rqs_block(u_ref[...], wt_ref[...], ht_ref[...], dt_ref[...])
    x_ref[...] = x
    ld_ref[...] = ld


def _tc_call(u, wt, ht, dt, interpret=False, n=None, row_offset=0):
    if n is None:
        n = u.shape[0]
    off_blocks = row_offset // _BR
    x, ld = pl.pallas_call(
        _tc_body,
        grid=(n // _BR,),
        in_specs=[
            pl.BlockSpec((_BR, _DIM), lambda i: (i + off_blocks, 0)),
            pl.BlockSpec((_K, _DIM), lambda i: (0, 0)),
            pl.BlockSpec((_K, _DIM), lambda i: (0, 0)),
            pl.BlockSpec((_K - 1, _DIM), lambda i: (0, 0)),
        ],
        out_specs=[
            pl.BlockSpec((_BR, _DIM), lambda i: (i, 0)),
            pl.BlockSpec((_BR, 1), lambda i: (i, 0)),
        ],
        out_shape=[
            jax.ShapeDtypeStruct((n, _DIM), u.dtype),
            jax.ShapeDtypeStruct((n, 1), u.dtype),
        ],
        interpret=interpret,
    )(u, wt, ht, dt)
    return x, ld.reshape(n)


_N_SC = 12288


@jax.jit
def kernel(u, w, h, d):
    wt, ht, dt = w.T, h.T, d.T
    tab = _prep_call(wt, ht, dt)
    x_sc, ld_sc = _make_sc_main(_N_SC)(u, tab)
    x_tc, ld_tc = _tc_call(u, wt, ht, dt, n=_N - _N_SC, row_offset=_N_SC)
    x = jnp.concatenate([x_sc, x_tc], axis=0)
    ld = jnp.concatenate([ld_sc, ld_tc], axis=0)
    return x, ld


# speedup vs baseline: 1.0279x; 1.0042x over previous
import functools

import jax
import jax.numpy as jnp
from jax import lax
from jax.experimental import pallas as pl
from jax.experimental.pallas import tpu as pltpu
from jax.experimental.pallas import tpu_sc as plsc

_B = 3.0
_K = 5
_DIM = 128
_N = 131072

_NC = 2
_NS = 16
_NW = _NC * _NS
_CH = 128
_ROWS_PER_W = _N // _NW

_TAB_ROWS = 32


def _knot_tables(wt, ht, dt):
    ew = jnp.exp(wt - jnp.max(wt, axis=0, keepdims=True))
    widths = ew / jnp.sum(ew, axis=0, keepdims=True) * (2.0 * _B)
    eh = jnp.exp(ht - jnp.max(ht, axis=0, keepdims=True))
    heights = eh / jnp.sum(eh, axis=0, keepdims=True) * (2.0 * _B)
    sp = jnp.maximum(dt, 0.0) + jnp.log1p(jnp.exp(-jnp.abs(dt)))

    negb = jnp.full((_DIM,), -_B, wt.dtype)
    cw = [negb]
    ch = [negb]
    for k in range(_K - 1):
        cw.append(cw[-1] + widths[k])
        ch.append(ch[-1] + heights[k])
    one = jnp.ones((_DIM,), wt.dtype)
    dv = [one] + [sp[k] for k in range(_K - 1)] + [one]
    return widths, heights, cw, ch, dv


def _prep_body(wt_ref, ht_ref, dt_ref, tab_ref):
    wt = wt_ref[...]
    ht = ht_ref[...]
    dt = dt_ref[...]
    widths, heights, cw, ch, dv = _knot_tables(wt, ht, dt)
    rows = []
    rows += [cw[k] for k in range(_K)]
    rows += [1.0 / widths[k] for k in range(_K)]
    rows += [ch[k] for k in range(_K)]
    rows += [heights[k] for k in range(_K)]
    rows += dv
    rows += [cw[k] for k in range(1, _K)]
    rows += [jnp.zeros((_DIM,), wt.dtype)] * 2
    tab_ref[...] = jnp.concatenate([r.reshape(1, _DIM) for r in rows], axis=0)


def _prep_call(wt, ht, dt):
    return pl.pallas_call(
        _prep_body,
        out_shape=jax.ShapeDtypeStruct((_TAB_ROWS, _DIM), wt.dtype),
    )(wt, ht, dt)


_LN2 = 0.6931471805599453
_LOG_C = (
    3.342326879394774e-08, 1.0000030986470898, -0.5000129330593671,
    0.33304812395026007, -0.24911210645450632, 0.20611785239613029,
    -0.18627697325403644, 0.11448435452423278,
)


def _sc_log(x):
    bits = lax.bitcast_convert_type(x, jnp.int32)
    e = lax.shift_right_arithmetic(bits, 23) - 127
    mbits = lax.bitwise_or(
        lax.bitwise_and(bits, 0x007FFFFF), 0x3F800000
    )
    m = lax.bitcast_convert_type(mbits, jnp.float32)
    big = m >= 1.4142135623730951
    m = jnp.where(big, m * 0.5, m)
    radj = jnp.where(big, _LN2, 0.0)
    t = m - 1.0
    p = jnp.float32(_LOG_C[7])
    for c in _LOG_C[6::-1]:
        p = p * t + c
    return e.astype(jnp.float32) * _LN2 + (radj + p)


def _sc_build_tab(wt_v, ht_v, dt_v, tab_v):
    for j in range(_DIM // 16):
        sl = pl.ds(16 * j, 16)
        for base_row, src in ((0, wt_v), (10, ht_v)):
            vk = [src[k, sl] for k in range(_K)]
            mx = jnp.maximum(
                jnp.maximum(jnp.maximum(vk[0], vk[1]),
                            jnp.maximum(vk[2], vk[3])), vk[4])
            ev = [jnp.exp(v - mx) for v in vk]
            tot = ev[0] + ev[1] + ev[2] + ev[3] + ev[4]
            rs = (2.0 * _B) / tot
            seg = [e * rs for e in ev]
            cum = jnp.full((16,), -_B, jnp.float32)
            for k in range(_K):
                tab_v[base_row + k, sl] = cum
                if base_row == 0:
                    tab_v[5 + k, sl] = 1.0 / seg[k]
                    if 1 <= k:
                        tab_v[26 + k - 1, sl] = cum
                else:
                    tab_v[15 + k, sl] = seg[k]
                cum = cum + seg[k]
        one = jnp.full((16,), 1.0, jnp.float32)
        tab_v[20, sl] = one
        tab_v[25, sl] = one
        for k in range(_K - 1):
            dvv = dt_v[k, sl]
            sp = jnp.maximum(dvv, 0.0) + _sc_log(1.0 + jnp.exp(-jnp.abs(dvv)))
            tab_v[21 + k, sl] = sp


def _sc_body(u_hbm, wt_hbm, ht_hbm, dt_hbm, x_hbm, ld_hbm,
             tab_v, wt_v, ht_v, dt_v, ubuf, xbuf, ldbuf, accbuf,
             *, rows_per_w):
    cid = lax.axis_index("c")
    sid = lax.axis_index("s")
    wid = sid * _NC + cid
    pltpu.sync_copy(wt_hbm, wt_v)
    pltpu.sync_copy(ht_hbm, ht_v)
    pltpu.sync_copy(dt_hbm, dt_v)
    _sc_build_tab(wt_v, ht_v, dt_v, tab_v)
    iota16 = lax.iota(jnp.int32, 16)

    def row_body(r, _):
        acc = jnp.zeros((16,), jnp.float32)
        for j in range(_DIM // 16):
            off = 16 * j
            u_v = ubuf[r, pl.ds(off, 16)]
            uc = jnp.clip(u_v, -_B, _B)
            inside = jnp.abs(u_v) <= _B
            binv = jnp.zeros((16,), jnp.int32)
            for k in range(_K - 1):
                ck = tab_v[26 + k, pl.ds(off, 16)]
                binv = binv + (uc >= ck).astype(jnp.int32)
            dimv = iota16 + off
            xk = plsc.load_gather(tab_v, [binv, dimv])
            rw = plsc.load_gather(tab_v, [binv + 5, dimv])
            yk = plsc.load_gather(tab_v, [binv + 10, dimv])
            hk = plsc.load_gather(tab_v, [binv + 15, dimv])
            dk = plsc.load_gather(tab_v, [binv + 20, dimv])
            dk1 = plsc.load_gather(tab_v, [binv + 21, dimv])
            theta = (uc - xk) * rw
            s = hk * rw
            omt = 1.0 - theta
            t1m = theta * omt
            denom = s + (dk + dk1 - 2.0 * s) * t1m
            rden = 1.0 / denom
            th2 = theta * theta
            x_in = yk + hk * (s * th2 + dk * t1m) * rden
            num = s * s * (dk1 * th2 + 2.0 * s * t1m + dk * omt * omt)
            ld_el = _sc_log(num * rden * rden)
            xbuf[r, pl.ds(off, 16)] = jnp.where(inside, x_in, u_v)
            acc = acc + jnp.where(inside, ld_el, 0.0)
        accbuf[r, :] = acc
        return 0

    def chunk_body(ci, _):
        base = wid * rows_per_w + ci * _CH
        pltpu.sync_copy(u_hbm.at[pl.ds(base, _CH)], ubuf)
        lax.fori_loop(0, _CH, row_body, 0)
        for rg in range(_CH // 16):
            rows = iota16 + (16 * rg)
            total = jnp.zeros((16,), jnp.float32)
            for c in range(16):
                cols = jnp.full((16,), c, jnp.int32)
                total = total + plsc.load_gather(accbuf, [rows, cols])
            ldbuf[pl.ds(16 * rg, 16)] = total
        pltpu.sync_copy(xbuf, x_hbm.at[pl.ds(base, _CH)])
        pltpu.sync_copy(ldbuf, ld_hbm.at[pl.ds(base, _CH)])
        return 0

    lax.fori_loop(0, rows_per_w // _CH, chunk_body, 0)


@functools.cache
def _make_sc_main(n_rows):
    rows_per_w = n_rows // _NW
    body = functools.partial(_sc_body, rows_per_w=rows_per_w)
    return functools.partial(
        pl.kernel,
        mesh=plsc.VectorSubcoreMesh(core_axis_name="c", subcore_axis_name="s"),
        out_type=[
            jax.ShapeDtypeStruct((n_rows, _DIM), jnp.float32),
            jax.ShapeDtypeStruct((n_rows,), jnp.float32),
        ],
        scratch_types=[
            pltpu.VMEM((_TAB_ROWS, _DIM), jnp.float32),
            pltpu.VMEM((_K, _DIM), jnp.float32),
            pltpu.VMEM((_K, _DIM), jnp.float32),
            pltpu.VMEM((_K - 1, _DIM), jnp.float32),
            pltpu.VMEM((_CH, _DIM), jnp.float32),
            pltpu.VMEM((_CH, _DIM), jnp.float32),
            pltpu.VMEM((_CH,), jnp.float32),
            pltpu.VMEM((_CH, 16), jnp.float32),
        ],
        compiler_params=pltpu.CompilerParams(needs_layout_passes=False),
    )(body)


_BR = 4096


def _rqs_block(u, wt, ht, dt):
    widths, heights, cw, ch, dv = _knot_tables(wt, ht, dt)
    cw = cw + [jnp.full((_DIM,), _B, u.dtype)]
    ch = ch + [jnp.full((_DIM,), _B, u.dtype)]
    rw = [1.0 / widths[k] for k in range(_K)]
    hh = [heights[k] for k in range(_K)]

    inside = (u >= -_B) & (u <= _B)
    uc = jnp.clip(u, -_B, _B)

    xk = jnp.broadcast_to(cw[0], u.shape)
    rwk = jnp.broadcast_to(rw[0], u.shape)
    yk = jnp.broadcast_to(ch[0], u.shape)
    hk = jnp.broadcast_to(hh[0], u.shape)
    dk = jnp.broadcast_to(dv[0], u.shape)
    dk1 = jnp.broadcast_to(dv[1], u.shape)
    for k in range(1, _K):
        m = uc >= cw[k]
        xk = jnp.where(m, cw[k], xk)
        rwk = jnp.where(m, rw[k], rwk)
        yk = jnp.where(m, ch[k], yk)
        hk = jnp.where(m, hh[k], hk)
        dk = jnp.where(m, dv[k], dk)
        dk1 = jnp.where(m, dv[k + 1], dk1)

    s = hk * rwk
    theta = (uc - xk) * rwk
    omt = 1.0 - theta
    t1m = theta * omt
    denom = s + (dk1 + dk - 2.0 * s) * t1m
    rden = 1.0 / denom
    x_in = yk + hk * (s * theta * theta + dk * t1m) * rden
    num = s * s * (dk1 * theta * theta + 2.0 * s * t1m + dk * omt * omt)
    logd_in = jnp.log(num * rden * rden)
    x = jnp.where(inside, x_in, u)
    ld = jnp.where(inside, logd_in, 0.0)
    return x, jnp.sum(ld, axis=-1, keepdims=True)


def _tc_body(u_ref, wt_ref, ht_ref, dt_ref, x_ref, ld_ref):
    x, ld = _rqs_block(u_ref[...], wt_ref[...], ht_ref[...], dt_ref[...])
    x_ref[...] = x
    ld_ref[...] = ld


def _tc_call(u, wt, ht, dt, interpret=False, n=None, row_offset=0):
    if n is None:
        n = u.shape[0]
    off_blocks = row_offset // _BR
    x, ld = pl.pallas_call(
        _tc_body,
        grid=(n // _BR,),
        in_specs=[
            pl.BlockSpec((_BR, _DIM), lambda i: (i + off_blocks, 0)),
            pl.BlockSpec((_K, _DIM), lambda i: (0, 0)),
            pl.BlockSpec((_K, _DIM), lambda i: (0, 0)),
            pl.BlockSpec((_K - 1, _DIM), lambda i: (0, 0)),
        ],
        out_specs=[
            pl.BlockSpec((_BR, _DIM), lambda i: (i, 0)),
            pl.BlockSpec((_BR, 1), lambda i: (i, 0)),
        ],
        out_shape=[
            jax.ShapeDtypeStruct((n, _DIM), u.dtype),
            jax.ShapeDtypeStruct((n, 1), u.dtype),
        ],
        interpret=interpret,
    )(u, wt, ht, dt)
    return x, ld.reshape(n)


_N_SC = 12288


@jax.jit
def kernel(u, w, h, d):
    wt, ht, dt = w.T, h.T, d.T
    x_sc, ld_sc = _make_sc_main(_N_SC)(u, wt, ht, dt)
    x_tc, ld_tc = _tc_call(u, wt, ht, dt, n=_N - _N_SC, row_offset=_N_SC)
    x = jnp.concatenate([x_sc, x_tc], axis=0)
    ld = jnp.concatenate([ld_sc, ld_tc], axis=0)
    return x, ld


# speedup vs baseline: 1.2264x; 1.1931x over previous
import functools

import jax
import jax.numpy as jnp
from jax import lax
from jax.experimental import pallas as pl
from jax.experimental.pallas import tpu as pltpu
from jax.experimental.pallas import tpu_sc as plsc

_B = 3.0
_K = 5
_DIM = 128
_N = 131072

_NC = 2
_NS = 16
_NW = _NC * _NS
_CH = 128
_ROWS_PER_W = _N // _NW

_TAB_ROWS = 32


def _knot_tables(wt, ht, dt):
    ew = jnp.exp(wt - jnp.max(wt, axis=0, keepdims=True))
    widths = ew / jnp.sum(ew, axis=0, keepdims=True) * (2.0 * _B)
    eh = jnp.exp(ht - jnp.max(ht, axis=0, keepdims=True))
    heights = eh / jnp.sum(eh, axis=0, keepdims=True) * (2.0 * _B)
    sp = jnp.maximum(dt, 0.0) + jnp.log1p(jnp.exp(-jnp.abs(dt)))

    negb = jnp.full((_DIM,), -_B, wt.dtype)
    cw = [negb]
    ch = [negb]
    for k in range(_K - 1):
        cw.append(cw[-1] + widths[k])
        ch.append(ch[-1] + heights[k])
    one = jnp.ones((_DIM,), wt.dtype)
    dv = [one] + [sp[k] for k in range(_K - 1)] + [one]
    return widths, heights, cw, ch, dv


def _prep_body(wt_ref, ht_ref, dt_ref, tab_ref):
    wt = wt_ref[...]
    ht = ht_ref[...]
    dt = dt_ref[...]
    widths, heights, cw, ch, dv = _knot_tables(wt, ht, dt)
    rows = []
    rows += [cw[k] for k in range(_K)]
    rows += [1.0 / widths[k] for k in range(_K)]
    rows += [ch[k] for k in range(_K)]
    rows += [heights[k] for k in range(_K)]
    rows += dv
    rows += [cw[k] for k in range(1, _K)]
    rows += [jnp.zeros((_DIM,), wt.dtype)] * 2
    tab_ref[...] = jnp.concatenate([r.reshape(1, _DIM) for r in rows], axis=0)


def _prep_call(wt, ht, dt):
    return pl.pallas_call(
        _prep_body,
        out_shape=jax.ShapeDtypeStruct((_TAB_ROWS, _DIM), wt.dtype),
    )(wt, ht, dt)


_LN2 = 0.6931471805599453
_LOG_C = (
    3.342326879394774e-08, 1.0000030986470898, -0.5000129330593671,
    0.33304812395026007, -0.24911210645450632, 0.20611785239613029,
    -0.18627697325403644, 0.11448435452423278,
)


def _sc_log(x):
    bits = lax.bitcast_convert_type(x, jnp.int32)
    e = lax.shift_right_arithmetic(bits, 23) - 127
    mbits = lax.bitwise_or(
        lax.bitwise_and(bits, 0x007FFFFF), 0x3F800000
    )
    m = lax.bitcast_convert_type(mbits, jnp.float32)
    big = m >= 1.4142135623730951
    m = jnp.where(big, m * 0.5, m)
    radj = jnp.where(big, _LN2, 0.0)
    t = m - 1.0
    p = jnp.float32(_LOG_C[7])
    for c in _LOG_C[6::-1]:
        p = p * t + c
    return e.astype(jnp.float32) * _LN2 + (radj + p)


def _sc_build_tab(wt_v, ht_v, dt_v, tab_v):
    for j in range(_DIM // 16):
        sl = pl.ds(16 * j, 16)
        for base_row, src in ((0, wt_v), (10, ht_v)):
            vk = [src[k, sl] for k in range(_K)]
            mx = jnp.maximum(
                jnp.maximum(jnp.maximum(vk[0], vk[1]),
                            jnp.maximum(vk[2], vk[3])), vk[4])
            ev = [jnp.exp(v - mx) for v in vk]
            tot = ev[0] + ev[1] + ev[2] + ev[3] + ev[4]
            rs = (2.0 * _B) / tot
            seg = [e * rs for e in ev]
            cum = jnp.full((16,), -_B, jnp.float32)
            for k in range(_K):
                tab_v[base_row + k, sl] = cum
                if base_row == 0:
                    tab_v[5 + k, sl] = 1.0 / seg[k]
                    if 1 <= k:
                        tab_v[26 + k - 1, sl] = cum
                else:
                    tab_v[15 + k, sl] = seg[k]
                cum = cum + seg[k]
        one = jnp.full((16,), 1.0, jnp.float32)
        tab_v[20, sl] = one
        tab_v[25, sl] = one
        for k in range(_K - 1):
            dvv = dt_v[k, sl]
            sp = jnp.maximum(dvv, 0.0) + _sc_log(1.0 + jnp.exp(-jnp.abs(dvv)))
            tab_v[21 + k, sl] = sp


def _sc_body(u_hbm, wt_hbm, ht_hbm, dt_hbm, x_hbm, ld_hbm,
             tab_v, wt_v, ht_v, dt_v, ubuf, xbuf, ldbuf, accbuf,
             *, rows_per_w):
    cid = lax.axis_index("c")
    sid = lax.axis_index("s")
    wid = sid * _NC + cid
    pltpu.sync_copy(wt_hbm, wt_v)
    pltpu.sync_copy(ht_hbm, ht_v)
    pltpu.sync_copy(dt_hbm, dt_v)
    _sc_build_tab(wt_v, ht_v, dt_v, tab_v)
    iota16 = lax.iota(jnp.int32, 16)

    def row_body(r, _):
        acc = jnp.zeros((16,), jnp.float32)
        for j in range(_DIM // 16):
            off = 16 * j
            u_v = ubuf[r, pl.ds(off, 16)]
            uc = jnp.clip(u_v, -_B, _B)
            inside = jnp.abs(u_v) <= _B
            binv = jnp.zeros((16,), jnp.int32)
            for k in range(_K - 1):
                ck = tab_v[26 + k, pl.ds(off, 16)]
                binv = binv + (uc >= ck).astype(jnp.int32)
            dimv = iota16 + off
            xk = plsc.load_gather(tab_v, [binv, dimv])
            rw = plsc.load_gather(tab_v, [binv + 5, dimv])
            yk = plsc.load_gather(tab_v, [binv + 10, dimv])
            hk = plsc.load_gather(tab_v, [binv + 15, dimv])
            dk = plsc.load_gather(tab_v, [binv + 20, dimv])
            dk1 = plsc.load_gather(tab_v, [binv + 21, dimv])
            theta = (uc - xk) * rw
            s = hk * rw
            omt = 1.0 - theta
            t1m = theta * omt
            denom = s + (dk + dk1 - 2.0 * s) * t1m
            rden = 1.0 / denom
            th2 = theta * theta
            x_in = yk + hk * (s * th2 + dk * t1m) * rden
            num = s * s * (dk1 * th2 + 2.0 * s * t1m + dk * omt * omt)
            ld_el = _sc_log(num * rden * rden)
            xbuf[r, pl.ds(off, 16)] = jnp.where(inside, x_in, u_v)
            acc = acc + jnp.where(inside, ld_el, 0.0)
        accbuf[r, :] = acc
        return 0

    def chunk_body(ci, _):
        base = wid * rows_per_w + ci * _CH
        pltpu.sync_copy(u_hbm.at[pl.ds(base, _CH)], ubuf)
        lax.fori_loop(0, _CH, row_body, 0)
        for rg in range(_CH // 16):
            rows = iota16 + (16 * rg)
            total = jnp.zeros((16,), jnp.float32)
            for c in range(16):
                cols = jnp.full((16,), c, jnp.int32)
                total = total + plsc.load_gather(accbuf, [rows, cols])
            ldbuf[pl.ds(16 * rg, 16)] = total
        pltpu.sync_copy(xbuf, x_hbm.at[pl.ds(base, _CH)])
        pltpu.sync_copy(ldbuf, ld_hbm.at[pl.ds(base, _CH)])
        return 0

    lax.fori_loop(0, rows_per_w // _CH, chunk_body, 0)


@functools.cache
def _make_sc_main(n_rows):
    rows_per_w = n_rows // _NW
    body = functools.partial(_sc_body, rows_per_w=rows_per_w)
    return functools.partial(
        pl.kernel,
        mesh=plsc.VectorSubcoreMesh(core_axis_name="c", subcore_axis_name="s"),
        out_type=[
            jax.ShapeDtypeStruct((n_rows, _DIM), jnp.float32),
            jax.ShapeDtypeStruct((n_rows,), jnp.float32),
        ],
        scratch_types=[
            pltpu.VMEM((_TAB_ROWS, _DIM), jnp.float32),
            pltpu.VMEM((_K, _DIM), jnp.float32),
            pltpu.VMEM((_K, _DIM), jnp.float32),
            pltpu.VMEM((_K - 1, _DIM), jnp.float32),
            pltpu.VMEM((_CH, _DIM), jnp.float32),
            pltpu.VMEM((_CH, _DIM), jnp.float32),
            pltpu.VMEM((_CH,), jnp.float32),
            pltpu.VMEM((_CH, 16), jnp.float32),
        ],
        compiler_params=pltpu.CompilerParams(needs_layout_passes=False),
    )(body)


_BR = 4096


def _rqs_block(u, wt, ht, dt):
    widths, heights, cw, ch, dv = _knot_tables(wt, ht, dt)
    cw = cw + [jnp.full((_DIM,), _B, u.dtype)]
    ch = ch + [jnp.full((_DIM,), _B, u.dtype)]
    rw = [1.0 / widths[k] for k in range(_K)]
    hh = [heights[k] for k in range(_K)]

    inside = (u >= -_B) & (u <= _B)
    uc = jnp.clip(u, -_B, _B)

    xk = jnp.broadcast_to(cw[0], u.shape)
    rwk = jnp.broadcast_to(rw[0], u.shape)
    yk = jnp.broadcast_to(ch[0], u.shape)
    hk = jnp.broadcast_to(hh[0], u.shape)
    dk = jnp.broadcast_to(dv[0], u.shape)
    dk1 = jnp.broadcast_to(dv[1], u.shape)
    for k in range(1, _K):
        m = uc >= cw[k]
        xk = jnp.where(m, cw[k], xk)
        rwk = jnp.where(m, rw[k], rwk)
        yk = jnp.where(m, ch[k], yk)
        hk = jnp.where(m, hh[k], hk)
        dk = jnp.where(m, dv[k], dk)
        dk1 = jnp.where(m, dv[k + 1], dk1)

    s = hk * rwk
    theta = (uc - xk) * rwk
    omt = 1.0 - theta
    t1m = theta * omt
    denom = s + (dk1 + dk - 2.0 * s) * t1m
    rden = 1.0 / denom
    x_in = yk + hk * (s * theta * theta + dk * t1m) * rden
    num = s * s * (dk1 * theta * theta + 2.0 * s * t1m + dk * omt * omt)
    logd_in = jnp.log(num * rden * rden)
    x = jnp.where(inside, x_in, u)
    ld = jnp.where(inside, logd_in, 0.0)
    return x, jnp.sum(ld, axis=-1, keepdims=True)


def _tc_body(u_ref, wt_ref, ht_ref, dt_ref, x_ref, ld_ref):
    x, ld = _rqs_block(u_ref[...], wt_ref[...], ht_ref[...], dt_ref[...])
    x_ref[...] = x
    ld_ref[...] = ld


def _tc_call(u, wt, ht, dt, interpret=False, n=None, row_offset=0):
    if n is None:
        n = u.shape[0]
    off_blocks = row_offset // _BR
    x, ld = pl.pallas_call(
        _tc_body,
        grid=(n // _BR,),
        in_specs=[
            pl.BlockSpec((_BR, _DIM), lambda i: (i + off_blocks, 0)),
            pl.BlockSpec((_K, _DIM), lambda i: (0, 0)),
            pl.BlockSpec((_K, _DIM), lambda i: (0, 0)),
            pl.BlockSpec((_K - 1, _DIM), lambda i: (0, 0)),
        ],
        out_specs=[
            pl.BlockSpec((_BR, _DIM), lambda i: (i + off_blocks, 0)),
            pl.BlockSpec((_BR, 1), lambda i: (i + off_blocks, 0)),
        ],
        out_shape=[
            jax.ShapeDtypeStruct((n + row_offset, _DIM), u.dtype),
            jax.ShapeDtypeStruct((n + row_offset, 1), u.dtype),
        ],
        interpret=interpret,
    )(u, wt, ht, dt)
    return x, ld.reshape(n + row_offset)


_N_SC = 12288


@jax.jit
def kernel(u, w, h, d):
    wt, ht, dt = w.T, h.T, d.T
    x_sc, ld_sc = _make_sc_main(_N_SC)(u, wt, ht, dt)
    x_tc, ld_tc = _tc_call(u, wt, ht, dt, n=_N - _N_SC, row_offset=_N_SC)
    x = lax.dynamic_update_slice(x_tc, x_sc, (0, 0))
    ld = lax.dynamic_update_slice(ld_tc, ld_sc, (0,))
    return x, ld
